# Initial kernel scaffold; baseline (speedup 1.0000x reference)
import jax
import jax.numpy as jnp
from jax.experimental import pallas as pl


def kernel(feature, edge_index):
    raise NotImplementedError("write your pallas kernel here")


# speedup vs baseline: 6.8009x; 6.8009x over previous
import jax
import jax.numpy as jnp
from jax import lax
from jax.experimental import pallas as pl
from jax.experimental.pallas import tpu as pltpu
from jax.experimental.pallas import tpu_sc as plsc

N = 10000
NPAD = 10240
E = 320000
D = 128
DH = 64
K = 8
NC = 2
NS = 16
CHUNK = 128
NCHUNK = 157
EPW = NCHUNK * CHUNK
PAD_ROW = 10200
STRIPE = NPAD // NS
HALF = STRIPE // 2
DEGROWS = NPAD // 64


def _sc_body(feat, rowi, coli, out, u_hbm,
             idx_row, idx_col, buf, stripe, zeros, deg_l, degs, dinv,
             ident, acc_sh, deg_sh):
    c = lax.axis_index("c")
    s = lax.axis_index("s")
    r0 = s * STRIPE
    ubase = c * NPAD

    pltpu.sync_copy(rowi.at[s], idx_row)
    pltpu.sync_copy(coli.at[s], idx_col)

    zero16 = jnp.zeros((16,), jnp.float32)
    ones16 = jnp.ones((16,), jnp.float32)
    iota16 = lax.iota(jnp.int32, 16)

    def _zfill(i, _):
        zeros[i // 4, pl.ds((i % 4) * 16, 16)] = zero16
        return 0
    lax.fori_loop(0, 256, _zfill, 0)
    for t in range(2):
        for j in range(5):
            ident[t, pl.ds(j * 16, 16)] = t * 80 + j * 16 + iota16

    def _coff(i, _):
        sl = (i // 8, pl.ds((i % 8) * 16, 16))
        idx_col[sl] = idx_col[sl] + ubase
        return 0
    lax.fori_loop(0, NCHUNK * 8, _coff, 0)

    def _dzero(i, _):
        deg_l[i // 4, pl.ds((i % 4) * 16, 16)] = zero16
        return 0
    lax.fori_loop(0, DEGROWS * 4, _dzero, 0)
    pltpu.sync_copy(zeros.at[pl.ds(0, 10)],
                    deg_sh.at[pl.ds(s * 10, 10)])
    plsc.subcore_barrier()

    def _hist(i, _):
        rows = idx_row[i // 8, pl.ds((i % 8) * 16, 16)]
        plsc.addupdate_scatter(deg_l, [rows // 64, rows % 64], ones16)
        return 0
    lax.fori_loop(0, NCHUNK * 8, _hist, 0)

    for t in range(2):
        pltpu.sync_copy(deg_l.at[pl.ds(t * 80, 80)],
                        deg_sh.at[ident.at[t]], add=True)
    plsc.subcore_barrier()

    pltpu.sync_copy(deg_sh.at[pl.ds(s * 10, 10)], degs)

    def _dinv(i, _):
        d = degs[i // 4, pl.ds((i % 4) * 16, 16)]
        y = plsc.bitcast(jnp.int32(0x5F3759DF) - (plsc.bitcast(d, jnp.int32) >> 1),
                         jnp.float32)
        y = y * (1.5 - 0.5 * d * y * y)
        y = y * (1.5 - 0.5 * d * y * y)
        y = y * (1.5 - 0.5 * d * y * y)
        y = jnp.where(d > 0.5, y, 0.0)
        dinv[pl.ds(i * 16, 16)] = y
        return 0
    lax.fori_loop(0, STRIPE // 16, _dinv, 0)

    def _scale(h):
        def body(g, _):
            dv = dinv[pl.ds(h * HALF + g * 16, 16)]
            for j in range(16):
                m = dv[j]
                r = g * 16 + j
                for q in range(4):
                    stripe[r, pl.ds(q * 16, 16)] = (
                        stripe[r, pl.ds(q * 16, 16)] * m)
            return 0
        lax.fori_loop(0, HALF // 16, body, 0)

    def _zero_acc(h):
        for j in range(HALF // 64):
            pltpu.sync_copy(zeros,
                            acc_sh.at[pl.ds(r0 + h * HALF + j * 64, 64)])

    for h in range(2):
        hb = r0 + h * HALF

        @pl.when(s < NS - 1)
        def _(hb=hb):
            pltpu.sync_copy(feat.at[pl.ds(hb, HALF), pl.ds(c * DH, DH)], stripe)
            pltpu.sync_copy(stripe,
                            out.at[pl.ds(hb, HALF), 0, pl.ds(c * DH, DH)])

        @pl.when(s == NS - 1)
        def _(hb=hb, h=h):
            last = 80 if h else HALF
            pltpu.sync_copy(feat.at[pl.ds(hb, last), pl.ds(c * DH, DH)],
                            stripe.at[pl.ds(0, last)])
            pltpu.sync_copy(stripe.at[pl.ds(0, last)],
                            out.at[pl.ds(hb, last), 0, pl.ds(c * DH, DH)])

        _scale(h)
        pltpu.sync_copy(stripe, u_hbm.at[pl.ds(ubase + hb, HALF)])
        _zero_acc(h)
    plsc.subcore_barrier()

    for k in range(1, K + 1):
        def _edge(j, _):
            pltpu.sync_copy(u_hbm.at[idx_col.at[j]], buf)
            pltpu.sync_copy(buf, acc_sh.at[idx_row.at[j]], add=True)
            return 0
        lax.fori_loop(0, NCHUNK, _edge, 0)
        plsc.subcore_barrier()

        for h in range(2):
            hb = r0 + h * HALF
            pltpu.sync_copy(acc_sh.at[pl.ds(hb, HALF)], stripe)
            _zero_acc(h)
            _scale(h)

            @pl.when(s < NS - 1)
            def _(hb=hb, k=k):
                pltpu.sync_copy(stripe,
                                out.at[pl.ds(hb, HALF), k, pl.ds(c * DH, DH)])

            @pl.when(s == NS - 1)
            def _(hb=hb, h=h, k=k):
                last = 80 if h else HALF
                pltpu.sync_copy(stripe.at[pl.ds(0, last)],
                                out.at[pl.ds(hb, last), k, pl.ds(c * DH, DH)])

            if k < K:
                _scale(h)
                pltpu.sync_copy(stripe, u_hbm.at[pl.ds(ubase + hb, HALF)])
        plsc.subcore_barrier()


@jax.jit
def _lgcn_sc(feature, rowi, coli):
    mesh = plsc.VectorSubcoreMesh(core_axis_name="c", subcore_axis_name="s")
    f = pl.kernel(
        _sc_body,
        out_type=(
            jax.ShapeDtypeStruct((N, K + 1, D), jnp.float32),
            jax.ShapeDtypeStruct((NC * NPAD, DH), jnp.float32),
        ),
        mesh=mesh,
        compiler_params=pltpu.CompilerParams(
            use_tc_tiling_on_sc=False, needs_layout_passes=False),
        scratch_types=[
            pltpu.VMEM((NCHUNK, CHUNK), jnp.int32),
            pltpu.VMEM((NCHUNK, CHUNK), jnp.int32),
            pltpu.VMEM((CHUNK, DH), jnp.float32),
            pltpu.VMEM((HALF, DH), jnp.float32),
            pltpu.VMEM((64, 64), jnp.float32),
            pltpu.VMEM((DEGROWS, 64), jnp.float32),
            pltpu.VMEM((STRIPE // 64, 64), jnp.float32),
            pltpu.VMEM((STRIPE,), jnp.float32),
            pltpu.VMEM((2, 80), jnp.int32),
            pltpu.VMEM_SHARED((NPAD, DH), jnp.float32),
            pltpu.VMEM_SHARED((DEGROWS, 64), jnp.float32),
        ],
    )
    return f(feature, rowi, coli)


def kernel(feature, edge_index):
    row = edge_index[0].astype(jnp.int32)
    col = edge_index[1].astype(jnp.int32)
    pad = NS * EPW - E
    row = jnp.concatenate([row, jnp.full((pad,), PAD_ROW, jnp.int32)])
    col = jnp.concatenate([col, jnp.zeros((pad,), jnp.int32)])
    rowi = row.reshape(NS, NCHUNK, CHUNK)
    coli = col.reshape(NS, NCHUNK, CHUNK)
    out, _ = _lgcn_sc(feature, rowi, coli)
    return out.reshape(N, (K + 1) * D)
